# baseline (speedup 1.0000x reference)
import functools

import jax
import jax.numpy as jnp
from jax import lax
from jax.experimental import pallas as pl
from jax.experimental.pallas import tpu as pltpu
from jax.experimental.pallas import tpu_sc as plsc

N = 10000
IN_DIM = 256
HID = 32
LATENT = 64

NC = 2
NS = 16
NW = NC * NS
K = 128
GATHER_W = 128
DEG_W = 128
N_PAD = 10112
ROWS_PER_TILE = N_PAD // NS
DUMMY = N

_ROW_BLK = 2000
_GRID = N // _ROW_BLK

_sc_mesh = plsc.VectorSubcoreMesh(
    core_axis_name="c", subcore_axis_name="s", num_cores=NC, num_subcores=NS
)


def _matmul_body(x_ref, w_ref, h_ref):
    h_ref[...] = jnp.dot(x_ref[...], w_ref[...],
                         preferred_element_type=jnp.float32)


def _tc_matmul(x, w):
    return pl.pallas_call(
        _matmul_body,
        grid=(_GRID,),
        in_specs=[pl.BlockSpec((_ROW_BLK, IN_DIM), lambda i: (i, 0)),
                  pl.BlockSpec((IN_DIM, HID), lambda i: (0, 0))],
        out_specs=pl.BlockSpec((_ROW_BLK, HID), lambda i: (i, 0)),
        out_shape=jax.ShapeDtypeStruct((N, HID), jnp.float32),
    )(x, w)


def _sc_degree(dst3, ones_rows, zeros_init):
    chunks = dst3.shape[1]

    @functools.partial(
        pl.kernel,
        out_type=jax.ShapeDtypeStruct((NC, N_PAD, DEG_W), jnp.float32),
        mesh=_sc_mesh,
        scratch_types=[
            pltpu.VMEM((chunks, K), jnp.int32),
            pltpu.VMEM((K, DEG_W), jnp.float32),
            pltpu.VMEM_SHARED((N_PAD, DEG_W), jnp.float32),
        ],
    )
    def k(dst_hbm, ones_hbm, zero_hbm, out_hbm, dst_v, ones_v, deg_sh):
        cid = lax.axis_index("c")
        sid = lax.axis_index("s")
        tid = cid * NS + sid
        pltpu.sync_copy(dst_hbm.at[tid], dst_v)
        pltpu.sync_copy(ones_hbm, ones_v)
        pltpu.sync_copy(zero_hbm, deg_sh.at[pl.ds(sid * ROWS_PER_TILE,
                                                  ROWS_PER_TILE)])
        plsc.subcore_barrier()

        @pl.loop(0, chunks)
        def _(j):
            pltpu.sync_copy(ones_v, deg_sh.at[dst_v.at[j]], add=True)

        plsc.subcore_barrier()
        pltpu.sync_copy(
            deg_sh.at[pl.ds(sid * ROWS_PER_TILE, ROWS_PER_TILE)],
            out_hbm.at[cid, pl.ds(sid * ROWS_PER_TILE, ROWS_PER_TILE)],
        )

    return k(dst3, ones_rows, zeros_init)


def _scale_body(h_ref, d0_ref, d1_ref, hp_ref):
    deg = d0_ref[:, 0:1] + d1_ref[:, 0:1] + 1.0
    hp_ref[...] = h_ref[...] * lax.rsqrt(deg)


def _tc_scale(h, d0, d1):
    return pl.pallas_call(
        _scale_body,
        grid=(_GRID,),
        in_specs=[pl.BlockSpec((_ROW_BLK, HID), lambda i: (i, 0)),
                  pl.BlockSpec((_ROW_BLK, 16), lambda i: (i, 0)),
                  pl.BlockSpec((_ROW_BLK, 16), lambda i: (i, 0))],
        out_specs=pl.BlockSpec((_ROW_BLK, HID), lambda i: (i, 0)),
        out_shape=jax.ShapeDtypeStruct((N, HID), jnp.float32),
    )(h, d0, d1)


def _sc_scatter(src3, dst3, hp_pad, zeros_init):
    chunks = src3.shape[1]

    NBUF = 2

    @functools.partial(
        pl.kernel,
        out_type=jax.ShapeDtypeStruct((NC, N_PAD, GATHER_W), jnp.float32),
        mesh=_sc_mesh,
        scratch_types=[
            pltpu.VMEM((chunks, K), jnp.int32),
            pltpu.VMEM((chunks, K), jnp.int32),
        ] + [pltpu.VMEM((K, GATHER_W), jnp.float32) for _ in range(NBUF)]
          + [pltpu.VMEM_SHARED((N_PAD, GATHER_W), jnp.float32)]
          + [pltpu.SemaphoreType.DMA for _ in range(NBUF)],
    )
    def k(src_hbm, dst_hbm, hp_hbm, zero_hbm, out_hbm,
          src_v, dst_v, *rest):
        rows = rest[:NBUF]
        acc_sh = rest[NBUF]
        sems = rest[NBUF + 1:]
        cid = lax.axis_index("c")
        sid = lax.axis_index("s")
        tid = cid * NS + sid
        pltpu.sync_copy(src_hbm.at[tid], src_v)
        pltpu.sync_copy(dst_hbm.at[tid], dst_v)
        pltpu.sync_copy(zero_hbm, acc_sh.at[pl.ds(sid * ROWS_PER_TILE,
                                                  ROWS_PER_TILE)])
        plsc.subcore_barrier()

        for p in range(NBUF - 1):
            pltpu.async_copy(hp_hbm.at[src_v.at[p]], rows[p], sems[p])

        @pl.loop(0, chunks, step=NBUF)
        def _(j):
            for b in range(NBUF):
                jb = j + b
                nxt = (b + NBUF - 1) % NBUF

                @pl.when(jb + NBUF - 1 < chunks)
                def _():
                    pltpu.async_copy(hp_hbm.at[src_v.at[jb + NBUF - 1]],
                                     rows[nxt], sems[nxt])

                pltpu.make_async_copy(hp_hbm.at[src_v.at[jb]],
                                      rows[b], sems[b]).wait()
                pltpu.sync_copy(rows[b], acc_sh.at[dst_v.at[jb]], add=True)

        plsc.subcore_barrier()
        pltpu.sync_copy(
            acc_sh.at[pl.ds(sid * ROWS_PER_TILE, ROWS_PER_TILE)],
            out_hbm.at[cid, pl.ds(sid * ROWS_PER_TILE, ROWS_PER_TILE)],
        )

    return k(src3, dst3, hp_pad, zeros_init)


def _final_body(h_ref, d0_ref, d1_ref, a0_ref, a1_ref, bg_ref,
                wm_ref, bm_ref, wl_ref, bl_ref, mu_ref, lv_ref, sacc):
    i = pl.program_id(0)
    dis = lax.rsqrt(d0_ref[:, 0:1] + d1_ref[:, 0:1] + 1.0)
    pre = dis * (a0_ref[...] + a1_ref[...]) + dis * dis * h_ref[...] + bg_ref[...]
    psum = jnp.sum(jnp.maximum(pre, 0.0), axis=0, keepdims=True)

    @pl.when(i == 0)
    def _():
        sacc[...] = psum

    @pl.when(i > 0)
    def _():
        sacc[...] += psum

    @pl.when(i == _GRID - 1)
    def _():
        pooled = sacc[...] * (1.0 / N)
        mu_ref[...] = jnp.dot(pooled, wm_ref[...],
                              preferred_element_type=jnp.float32) + bm_ref[...]
        lv_ref[...] = jnp.dot(pooled, wl_ref[...],
                              preferred_element_type=jnp.float32) + bl_ref[...]


def _tc_final(h, d0, d1, a0, a1, bg, wm, bm, wl, bl):
    return pl.pallas_call(
        _final_body,
        grid=(_GRID,),
        in_specs=[pl.BlockSpec((_ROW_BLK, HID), lambda i: (i, 0)),
                  pl.BlockSpec((_ROW_BLK, 16), lambda i: (i, 0)),
                  pl.BlockSpec((_ROW_BLK, 16), lambda i: (i, 0)),
                  pl.BlockSpec((_ROW_BLK, HID), lambda i: (i, 0)),
                  pl.BlockSpec((_ROW_BLK, HID), lambda i: (i, 0)),
                  pl.BlockSpec((1, HID), lambda i: (0, 0)),
                  pl.BlockSpec((HID, LATENT), lambda i: (0, 0)),
                  pl.BlockSpec((1, LATENT), lambda i: (0, 0)),
                  pl.BlockSpec((HID, LATENT), lambda i: (0, 0)),
                  pl.BlockSpec((1, LATENT), lambda i: (0, 0))],
        out_specs=[pl.BlockSpec((1, LATENT), lambda i: (0, 0)),
                   pl.BlockSpec((1, LATENT), lambda i: (0, 0))],
        out_shape=[jax.ShapeDtypeStruct((1, LATENT), jnp.float32),
                   jax.ShapeDtypeStruct((1, LATENT), jnp.float32)],
        scratch_shapes=[pltpu.VMEM((1, HID), jnp.float32)],
    )(h, d0, d1, a0, a1, bg, wm, bm, wl, bl)


def kernel(x, edge_index, W_gcn, b_gcn, W_mu, b_mu, W_lv, b_lv):
    e = edge_index.shape[1]
    e_pad = ((e + NW * K - 1) // (NW * K)) * (NW * K)
    chunks = e_pad // (NW * K)

    pad = jnp.full((e_pad - e,), DUMMY, jnp.int32)
    src3 = jnp.concatenate([edge_index[0], pad]).reshape(NW, chunks, K)
    dst3 = jnp.concatenate([edge_index[1], pad]).reshape(NW, chunks, K)

    ones_rows = jnp.ones((K, DEG_W), jnp.float32)
    zerosd = jnp.zeros((ROWS_PER_TILE, DEG_W), jnp.float32)
    zerosw = jnp.zeros((ROWS_PER_TILE, GATHER_W), jnp.float32)

    h = _tc_matmul(x, W_gcn)
    degp = _sc_degree(dst3, ones_rows, zerosd)
    d0 = degp[0, :N, :16]
    d1 = degp[1, :N, :16]

    hp = _tc_scale(h, d0, d1)
    hp_pad = jnp.pad(hp, ((0, N_PAD - N), (0, GATHER_W - HID)))
    accp = _sc_scatter(src3, dst3, hp_pad, zerosw)
    a0 = accp[0, :N, :HID]
    a1 = accp[1, :N, :HID]

    mu2, lv2 = _tc_final(h, d0, d1, a0, a1,
                         b_gcn.reshape(1, HID), W_mu, b_mu.reshape(1, LATENT),
                         W_lv, b_lv.reshape(1, LATENT))
    return mu2.reshape(LATENT), lv2.reshape(LATENT)


# speedup vs baseline: 1.9560x; 1.9560x over previous
import functools

import jax
import jax.numpy as jnp
from jax import lax
from jax.experimental import pallas as pl
from jax.experimental.pallas import tpu as pltpu
from jax.experimental.pallas import tpu_sc as plsc

N = 10000
IN_DIM = 256
HID = 32
LATENT = 64

NC = 2
NS = 16
NW = NC * NS
K = 128
GATHER_W = 128
DEG_W = 128
N_PAD = 10112
ROWS_PER_TILE = N_PAD // NS
DUMMY = N

_ROW_BLK = 2000
_GRID = N // _ROW_BLK

_sc_mesh = plsc.VectorSubcoreMesh(
    core_axis_name="c", subcore_axis_name="s", num_cores=NC, num_subcores=NS
)


def _matmul_body(x_ref, w_ref, h_ref):
    h_ref[...] = jnp.dot(x_ref[...], w_ref[...],
                         preferred_element_type=jnp.float32)


def _tc_matmul(x, w):
    return pl.pallas_call(
        _matmul_body,
        grid=(_GRID,),
        in_specs=[pl.BlockSpec((_ROW_BLK, IN_DIM), lambda i: (i, 0)),
                  pl.BlockSpec((IN_DIM, HID), lambda i: (0, 0))],
        out_specs=pl.BlockSpec((_ROW_BLK, HID), lambda i: (i, 0)),
        out_shape=jax.ShapeDtypeStruct((N, HID), jnp.float32),
    )(x, w)


def _sc_degree(dst3, ones_rows, zeros_init):
    chunks = dst3.shape[1]

    @functools.partial(
        pl.kernel,
        out_type=jax.ShapeDtypeStruct((NC, N_PAD, DEG_W), jnp.float32),
        mesh=_sc_mesh,
        scratch_types=[
            pltpu.VMEM((chunks, K), jnp.int32),
            pltpu.VMEM((K, DEG_W), jnp.float32),
            pltpu.VMEM_SHARED((N_PAD, DEG_W), jnp.float32),
        ],
    )
    def k(dst_hbm, ones_hbm, zero_hbm, out_hbm, dst_v, ones_v, deg_sh):
        cid = lax.axis_index("c")
        sid = lax.axis_index("s")
        tid = cid * NS + sid
        pltpu.sync_copy(dst_hbm.at[tid], dst_v)
        pltpu.sync_copy(ones_hbm, ones_v)
        pltpu.sync_copy(zero_hbm, deg_sh.at[pl.ds(sid * ROWS_PER_TILE,
                                                  ROWS_PER_TILE)])
        plsc.subcore_barrier()

        @pl.loop(0, chunks)
        def _(j):
            pltpu.sync_copy(ones_v, deg_sh.at[dst_v.at[j]], add=True)

        plsc.subcore_barrier()
        pltpu.sync_copy(
            deg_sh.at[pl.ds(sid * ROWS_PER_TILE, ROWS_PER_TILE)],
            out_hbm.at[cid, pl.ds(sid * ROWS_PER_TILE, ROWS_PER_TILE)],
        )

    return k(dst3, ones_rows, zeros_init)


def _scale_body(h_ref, d0_ref, d1_ref, hp_ref):
    deg = d0_ref[:, 0:1] + d1_ref[:, 0:1] + 1.0
    hp_ref[...] = h_ref[...] * lax.rsqrt(deg)


def _tc_scale(h, d0, d1):
    return pl.pallas_call(
        _scale_body,
        grid=(_GRID,),
        in_specs=[pl.BlockSpec((_ROW_BLK, HID), lambda i: (i, 0)),
                  pl.BlockSpec((_ROW_BLK, 16), lambda i: (i, 0)),
                  pl.BlockSpec((_ROW_BLK, 16), lambda i: (i, 0))],
        out_specs=pl.BlockSpec((_ROW_BLK, HID), lambda i: (i, 0)),
        out_shape=jax.ShapeDtypeStruct((N, HID), jnp.float32),
    )(h, d0, d1)


def _sc_scatter(src3, dst3, hp_pad, zeros_init):
    chunks = src3.shape[1]

    NBUF = 2

    @functools.partial(
        pl.kernel,
        out_type=jax.ShapeDtypeStruct((NC, N_PAD, GATHER_W), jnp.float32),
        mesh=_sc_mesh,
        scratch_types=[
            pltpu.VMEM((chunks, K), jnp.int32),
            pltpu.VMEM((chunks, K), jnp.int32),
        ] + [pltpu.VMEM((K, GATHER_W), jnp.float32) for _ in range(NBUF)]
          + [pltpu.VMEM_SHARED((N_PAD, GATHER_W), jnp.float32)]
          + [pltpu.SemaphoreType.DMA for _ in range(NBUF)],
    )
    def k(src_hbm, dst_hbm, hp_hbm, zero_hbm, out_hbm,
          src_v, dst_v, *rest):
        rows = rest[:NBUF]
        acc_sh = rest[NBUF]
        sems = rest[NBUF + 1:]
        cid = lax.axis_index("c")
        sid = lax.axis_index("s")
        tid = cid * NS + sid
        pltpu.sync_copy(src_hbm.at[tid], src_v)
        pltpu.sync_copy(dst_hbm.at[tid], dst_v)
        pltpu.sync_copy(zero_hbm, acc_sh.at[pl.ds(sid * ROWS_PER_TILE,
                                                  ROWS_PER_TILE)])
        plsc.subcore_barrier()

        for p in range(NBUF - 1):
            pltpu.async_copy(hp_hbm.at[src_v.at[p]], rows[p], sems[p])

        @pl.loop(0, chunks, step=NBUF)
        def _(j):
            for b in range(NBUF):
                jb = j + b
                nxt = (b + NBUF - 1) % NBUF

                @pl.when(jb + NBUF - 1 < chunks)
                def _():
                    pltpu.async_copy(hp_hbm.at[src_v.at[jb + NBUF - 1]],
                                     rows[nxt], sems[nxt])

                pltpu.make_async_copy(hp_hbm.at[src_v.at[jb]],
                                      rows[b], sems[b]).wait()
                pltpu.sync_copy(rows[b], acc_sh.at[dst_v.at[jb]], add=True)

        plsc.subcore_barrier()
        pltpu.sync_copy(
            acc_sh.at[pl.ds(sid * ROWS_PER_TILE, ROWS_PER_TILE)],
            out_hbm.at[cid, pl.ds(sid * ROWS_PER_TILE, ROWS_PER_TILE)],
        )

    return k(src3, dst3, hp_pad, zeros_init)


def _final_body(h_ref, d0_ref, d1_ref, a0_ref, a1_ref, bg_ref,
                wm_ref, bm_ref, wl_ref, bl_ref, mu_ref, lv_ref, sacc):
    i = pl.program_id(0)
    dis = lax.rsqrt(d0_ref[:, 0:1] + d1_ref[:, 0:1] + 1.0)
    pre = dis * (a0_ref[...] + a1_ref[...]) + dis * dis * h_ref[...] + bg_ref[...]
    psum = jnp.sum(jnp.maximum(pre, 0.0), axis=0, keepdims=True)

    @pl.when(i == 0)
    def _():
        sacc[...] = psum

    @pl.when(i > 0)
    def _():
        sacc[...] += psum

    @pl.when(i == _GRID - 1)
    def _():
        pooled = sacc[...] * (1.0 / N)
        mu_ref[...] = jnp.dot(pooled, wm_ref[...],
                              preferred_element_type=jnp.float32) + bm_ref[...]
        lv_ref[...] = jnp.dot(pooled, wl_ref[...],
                              preferred_element_type=jnp.float32) + bl_ref[...]


def _tc_final(h, d0, d1, a0, a1, bg, wm, bm, wl, bl):
    return pl.pallas_call(
        _final_body,
        grid=(_GRID,),
        in_specs=[pl.BlockSpec((_ROW_BLK, HID), lambda i: (i, 0)),
                  pl.BlockSpec((_ROW_BLK, 16), lambda i: (i, 0)),
                  pl.BlockSpec((_ROW_BLK, 16), lambda i: (i, 0)),
                  pl.BlockSpec((_ROW_BLK, HID), lambda i: (i, 0)),
                  pl.BlockSpec((_ROW_BLK, HID), lambda i: (i, 0)),
                  pl.BlockSpec((1, HID), lambda i: (0, 0)),
                  pl.BlockSpec((HID, LATENT), lambda i: (0, 0)),
                  pl.BlockSpec((1, LATENT), lambda i: (0, 0)),
                  pl.BlockSpec((HID, LATENT), lambda i: (0, 0)),
                  pl.BlockSpec((1, LATENT), lambda i: (0, 0))],
        out_specs=[pl.BlockSpec((1, LATENT), lambda i: (0, 0)),
                   pl.BlockSpec((1, LATENT), lambda i: (0, 0))],
        out_shape=[jax.ShapeDtypeStruct((1, LATENT), jnp.float32),
                   jax.ShapeDtypeStruct((1, LATENT), jnp.float32)],
        scratch_shapes=[pltpu.VMEM((1, HID), jnp.float32)],
    )(h, d0, d1, a0, a1, bg, wm, bm, wl, bl)


def kernel(x, edge_index, W_gcn, b_gcn, W_mu, b_mu, W_lv, b_lv):
    e = edge_index.shape[1]
    e1 = ((e + NW - 1) // NW) * NW
    pw = e1 // NW
    chunks = (pw + K - 1) // K
    cols_pad = chunks * K - pw

    tail1 = jnp.full((e1 - e,), DUMMY, jnp.int32)
    src2 = jnp.concatenate([edge_index[0], tail1]).reshape(NW, pw)
    dst2 = jnp.concatenate([edge_index[1], tail1]).reshape(NW, pw)
    dummies = (DUMMY + (jnp.arange(NW * cols_pad, dtype=jnp.int32)
                        % (N_PAD - N))).reshape(NW, cols_pad)
    src3 = jnp.concatenate([src2, dummies], axis=1).reshape(NW, chunks, K)
    dst3 = jnp.concatenate([dst2, dummies], axis=1).reshape(NW, chunks, K)

    ones_rows = jnp.ones((K, DEG_W), jnp.float32)
    zerosd = jnp.zeros((ROWS_PER_TILE, DEG_W), jnp.float32)
    zerosw = jnp.zeros((ROWS_PER_TILE, GATHER_W), jnp.float32)

    h = _tc_matmul(x, W_gcn)
    degp = _sc_degree(dst3, ones_rows, zerosd)
    d0 = degp[0, :N, :16]
    d1 = degp[1, :N, :16]

    hp = _tc_scale(h, d0, d1)
    hp_pad = jnp.pad(hp, ((0, N_PAD - N), (0, GATHER_W - HID)))
    accp = _sc_scatter(src3, dst3, hp_pad, zerosw)
    a0 = accp[0, :N, :HID]
    a1 = accp[1, :N, :HID]

    mu2, lv2 = _tc_final(h, d0, d1, a0, a1,
                         b_gcn.reshape(1, HID), W_mu, b_mu.reshape(1, LATENT),
                         W_lv, b_lv.reshape(1, LATENT))
    return mu2.reshape(LATENT), lv2.reshape(LATENT)


# speedup vs baseline: 1.9920x; 1.0184x over previous
import functools

import jax
import jax.numpy as jnp
from jax import lax
from jax.experimental import pallas as pl
from jax.experimental.pallas import tpu as pltpu
from jax.experimental.pallas import tpu_sc as plsc

N = 10000
IN_DIM = 256
HID = 32
LATENT = 64

NC = 2
NS = 16
NW = NC * NS
K = 128
GATHER_W = 128
DEG_W = 128
N_PAD = 10112
ROWS_PER_TILE = N_PAD // NS
DUMMY = N

_ROW_BLK = 2000
_GRID = N // _ROW_BLK

_sc_mesh = plsc.VectorSubcoreMesh(
    core_axis_name="c", subcore_axis_name="s", num_cores=NC, num_subcores=NS
)


def _matmul_body(x_ref, w_ref, h_ref):
    h_ref[...] = jnp.dot(x_ref[...], w_ref[...],
                         preferred_element_type=jnp.float32)


def _tc_matmul(x, w):
    return pl.pallas_call(
        _matmul_body,
        grid=(_GRID,),
        in_specs=[pl.BlockSpec((_ROW_BLK, IN_DIM), lambda i: (i, 0)),
                  pl.BlockSpec((IN_DIM, HID), lambda i: (0, 0))],
        out_specs=pl.BlockSpec((_ROW_BLK, HID), lambda i: (i, 0)),
        out_shape=jax.ShapeDtypeStruct((N, HID), jnp.float32),
    )(x, w)


def _sc_degree(dst3, ones_rows, zeros_init):
    chunks = dst3.shape[1]

    @functools.partial(
        pl.kernel,
        out_type=jax.ShapeDtypeStruct((NC, N_PAD, DEG_W), jnp.float32),
        mesh=_sc_mesh,
        scratch_types=[
            pltpu.VMEM((chunks, K), jnp.int32),
            pltpu.VMEM((K, DEG_W), jnp.float32),
            pltpu.VMEM_SHARED((N_PAD, DEG_W), jnp.float32),
        ],
    )
    def k(dst_hbm, ones_hbm, zero_hbm, out_hbm, dst_v, ones_v, deg_sh):
        cid = lax.axis_index("c")
        sid = lax.axis_index("s")
        tid = cid * NS + sid
        pltpu.sync_copy(dst_hbm.at[tid], dst_v)
        pltpu.sync_copy(ones_hbm, ones_v)
        pltpu.sync_copy(zero_hbm, deg_sh.at[pl.ds(sid * ROWS_PER_TILE,
                                                  ROWS_PER_TILE)])
        plsc.subcore_barrier()

        @pl.loop(0, chunks)
        def _(j):
            pltpu.sync_copy(ones_v, deg_sh.at[dst_v.at[j]], add=True)

        plsc.subcore_barrier()
        pltpu.sync_copy(
            deg_sh.at[pl.ds(sid * ROWS_PER_TILE, ROWS_PER_TILE)],
            out_hbm.at[cid, pl.ds(sid * ROWS_PER_TILE, ROWS_PER_TILE)],
        )

    return k(dst3, ones_rows, zeros_init)


def _scale_body(h_ref, d0_ref, d1_ref, hp_ref):
    deg = d0_ref[:, 0:1] + d1_ref[:, 0:1] + 1.0
    scaled = h_ref[...] * lax.rsqrt(deg)
    hp_ref[...] = jnp.concatenate(
        [scaled, jnp.zeros((_ROW_BLK, GATHER_W - HID), jnp.float32)], axis=1)


def _tc_scale(h, d0, d1):
    return pl.pallas_call(
        _scale_body,
        grid=(_GRID,),
        in_specs=[pl.BlockSpec((_ROW_BLK, HID), lambda i: (i, 0)),
                  pl.BlockSpec((_ROW_BLK, 16), lambda i: (i, 0)),
                  pl.BlockSpec((_ROW_BLK, 16), lambda i: (i, 0))],
        out_specs=pl.BlockSpec((_ROW_BLK, GATHER_W), lambda i: (i, 0)),
        out_shape=jax.ShapeDtypeStruct((N_PAD, GATHER_W), jnp.float32),
    )(h, d0, d1)


def _sc_scatter(src3, dst3, hp_pad, zeros_init):
    chunks = src3.shape[1]

    NBUF = 2

    @functools.partial(
        pl.kernel,
        out_type=jax.ShapeDtypeStruct((NC, N_PAD, GATHER_W), jnp.float32),
        mesh=_sc_mesh,
        scratch_types=[
            pltpu.VMEM((chunks, K), jnp.int32),
            pltpu.VMEM((chunks, K), jnp.int32),
        ] + [pltpu.VMEM((K, GATHER_W), jnp.float32) for _ in range(NBUF)]
          + [pltpu.VMEM_SHARED((N_PAD, GATHER_W), jnp.float32)]
          + [pltpu.SemaphoreType.DMA for _ in range(NBUF)],
    )
    def k(src_hbm, dst_hbm, hp_hbm, zero_hbm, out_hbm,
          src_v, dst_v, *rest):
        rows = rest[:NBUF]
        acc_sh = rest[NBUF]
        sems = rest[NBUF + 1:]
        cid = lax.axis_index("c")
        sid = lax.axis_index("s")
        tid = cid * NS + sid
        pltpu.sync_copy(src_hbm.at[tid], src_v)
        pltpu.sync_copy(dst_hbm.at[tid], dst_v)
        pltpu.sync_copy(zero_hbm, acc_sh.at[pl.ds(sid * ROWS_PER_TILE,
                                                  ROWS_PER_TILE)])
        plsc.subcore_barrier()

        for p in range(NBUF - 1):
            pltpu.async_copy(hp_hbm.at[src_v.at[p]], rows[p], sems[p])

        @pl.loop(0, chunks, step=NBUF)
        def _(j):
            for b in range(NBUF):
                jb = j + b
                nxt = (b + NBUF - 1) % NBUF

                @pl.when(jb + NBUF - 1 < chunks)
                def _():
                    pltpu.async_copy(hp_hbm.at[src_v.at[jb + NBUF - 1]],
                                     rows[nxt], sems[nxt])

                pltpu.make_async_copy(hp_hbm.at[src_v.at[jb]],
                                      rows[b], sems[b]).wait()
                pltpu.sync_copy(rows[b], acc_sh.at[dst_v.at[jb]], add=True)

        plsc.subcore_barrier()
        pltpu.sync_copy(
            acc_sh.at[pl.ds(sid * ROWS_PER_TILE, ROWS_PER_TILE)],
            out_hbm.at[cid, pl.ds(sid * ROWS_PER_TILE, ROWS_PER_TILE)],
        )

    return k(src3, dst3, hp_pad, zeros_init)


def _final_body(h_ref, d0_ref, d1_ref, a0_ref, a1_ref, bg_ref,
                wm_ref, bm_ref, wl_ref, bl_ref, mu_ref, lv_ref, sacc):
    i = pl.program_id(0)
    dis = lax.rsqrt(d0_ref[:, 0:1] + d1_ref[:, 0:1] + 1.0)
    pre = dis * (a0_ref[...] + a1_ref[...]) + dis * dis * h_ref[...] + bg_ref[...]
    psum = jnp.sum(jnp.maximum(pre, 0.0), axis=0, keepdims=True)

    @pl.when(i == 0)
    def _():
        sacc[...] = psum

    @pl.when(i > 0)
    def _():
        sacc[...] += psum

    @pl.when(i == _GRID - 1)
    def _():
        pooled = sacc[...] * (1.0 / N)
        mu_ref[...] = jnp.dot(pooled, wm_ref[...],
                              preferred_element_type=jnp.float32) + bm_ref[...]
        lv_ref[...] = jnp.dot(pooled, wl_ref[...],
                              preferred_element_type=jnp.float32) + bl_ref[...]


def _tc_final(h, d0, d1, a0, a1, bg, wm, bm, wl, bl):
    return pl.pallas_call(
        _final_body,
        grid=(_GRID,),
        in_specs=[pl.BlockSpec((_ROW_BLK, HID), lambda i: (i, 0)),
                  pl.BlockSpec((_ROW_BLK, 16), lambda i: (i, 0)),
                  pl.BlockSpec((_ROW_BLK, 16), lambda i: (i, 0)),
                  pl.BlockSpec((_ROW_BLK, HID), lambda i: (i, 0)),
                  pl.BlockSpec((_ROW_BLK, HID), lambda i: (i, 0)),
                  pl.BlockSpec((1, HID), lambda i: (0, 0)),
                  pl.BlockSpec((HID, LATENT), lambda i: (0, 0)),
                  pl.BlockSpec((1, LATENT), lambda i: (0, 0)),
                  pl.BlockSpec((HID, LATENT), lambda i: (0, 0)),
                  pl.BlockSpec((1, LATENT), lambda i: (0, 0))],
        out_specs=[pl.BlockSpec((1, LATENT), lambda i: (0, 0)),
                   pl.BlockSpec((1, LATENT), lambda i: (0, 0))],
        out_shape=[jax.ShapeDtypeStruct((1, LATENT), jnp.float32),
                   jax.ShapeDtypeStruct((1, LATENT), jnp.float32)],
        scratch_shapes=[pltpu.VMEM((1, HID), jnp.float32)],
    )(h, d0, d1, a0, a1, bg, wm, bm, wl, bl)


def kernel(x, edge_index, W_gcn, b_gcn, W_mu, b_mu, W_lv, b_lv):
    e = edge_index.shape[1]
    e1 = ((e + NW - 1) // NW) * NW
    pw = e1 // NW
    chunks = (pw + K - 1) // K
    cols_pad = chunks * K - pw

    tail1 = jnp.full((e1 - e,), DUMMY, jnp.int32)
    src2 = jnp.concatenate([edge_index[0], tail1]).reshape(NW, pw)
    dst2 = jnp.concatenate([edge_index[1], tail1]).reshape(NW, pw)
    dummies = (DUMMY + (jnp.arange(NW * cols_pad, dtype=jnp.int32)
                        % (N_PAD - N))).reshape(NW, cols_pad)
    src3 = jnp.concatenate([src2, dummies], axis=1).reshape(NW, chunks, K)
    dst3 = jnp.concatenate([dst2, dummies], axis=1).reshape(NW, chunks, K)

    ones_rows = jnp.ones((K, DEG_W), jnp.float32)
    zerosd = jnp.zeros((ROWS_PER_TILE, DEG_W), jnp.float32)
    zerosw = jnp.zeros((ROWS_PER_TILE, GATHER_W), jnp.float32)

    h = _tc_matmul(x, W_gcn)
    degp = _sc_degree(dst3, ones_rows, zerosd)
    d0 = degp[0, :N, :16]
    d1 = degp[1, :N, :16]

    hp_pad = _tc_scale(h, d0, d1)
    accp = _sc_scatter(src3, dst3, hp_pad, zerosw)
    a0 = accp[0, :N, :HID]
    a1 = accp[1, :N, :HID]

    mu2, lv2 = _tc_final(h, d0, d1, a0, a1,
                         b_gcn.reshape(1, HID), W_mu, b_mu.reshape(1, LATENT),
                         W_lv, b_lv.reshape(1, LATENT))
    return mu2.reshape(LATENT), lv2.reshape(LATENT)


# speedup vs baseline: 2.1664x; 1.0875x over previous
import functools

import jax
import jax.numpy as jnp
from jax import lax
from jax.experimental import pallas as pl
from jax.experimental.pallas import tpu as pltpu
from jax.experimental.pallas import tpu_sc as plsc

N = 10000
IN_DIM = 256
HID = 32
LATENT = 64

NC = 2
NS = 16
NW = NC * NS
K = 128
GATHER_W = 128
DEG_W = 128
N_PAD = 10112
ROWS_PER_TILE = N_PAD // NS
DUMMY = N

_ROW_BLK = 2000
_GRID = N // _ROW_BLK

_sc_mesh = plsc.VectorSubcoreMesh(
    core_axis_name="c", subcore_axis_name="s", num_cores=NC, num_subcores=NS
)


def _matmul_body(x_ref, w_ref, h_ref):
    h_ref[...] = jnp.dot(x_ref[...], w_ref[...],
                         preferred_element_type=jnp.float32)


def _tc_matmul(x, w):
    return pl.pallas_call(
        _matmul_body,
        grid=(_GRID,),
        in_specs=[pl.BlockSpec((_ROW_BLK, IN_DIM), lambda i: (i, 0)),
                  pl.BlockSpec((IN_DIM, HID), lambda i: (0, 0))],
        out_specs=pl.BlockSpec((_ROW_BLK, HID), lambda i: (i, 0)),
        out_shape=jax.ShapeDtypeStruct((N, HID), jnp.float32),
    )(x, w)


def _sc_degree(dst3, ones_rows, zeros_init):
    chunks = dst3.shape[1]

    @functools.partial(
        pl.kernel,
        out_type=jax.ShapeDtypeStruct((NC, N_PAD, DEG_W), jnp.float32),
        mesh=_sc_mesh,
        scratch_types=[
            pltpu.VMEM((chunks, K), jnp.int32),
            pltpu.VMEM((K, DEG_W), jnp.float32),
            pltpu.VMEM_SHARED((N_PAD, DEG_W), jnp.float32),
        ],
    )
    def k(dst_hbm, ones_hbm, zero_hbm, out_hbm, dst_v, ones_v, deg_sh):
        cid = lax.axis_index("c")
        sid = lax.axis_index("s")
        tid = cid * NS + sid
        pltpu.sync_copy(dst_hbm.at[tid], dst_v)
        pltpu.sync_copy(ones_hbm, ones_v)
        pltpu.sync_copy(zero_hbm, deg_sh.at[pl.ds(sid * ROWS_PER_TILE,
                                                  ROWS_PER_TILE)])
        plsc.subcore_barrier()

        @pl.loop(0, chunks)
        def _(j):
            pltpu.sync_copy(ones_v, deg_sh.at[dst_v.at[j]], add=True)

        plsc.subcore_barrier()
        pltpu.sync_copy(
            deg_sh.at[pl.ds(sid * ROWS_PER_TILE, ROWS_PER_TILE)],
            out_hbm.at[cid, pl.ds(sid * ROWS_PER_TILE, ROWS_PER_TILE)],
        )

    return k(dst3, ones_rows, zeros_init)


def _scale_body(h_ref, deg_ref, hp_ref):
    deg = deg_ref[0, :, 0:1] + deg_ref[1, :, 0:1] + 1.0
    scaled = h_ref[...] * lax.rsqrt(deg)
    hp_ref[...] = jnp.concatenate(
        [scaled, jnp.zeros((_ROW_BLK, GATHER_W - HID), jnp.float32)], axis=1)


def _tc_scale(h, degp):
    return pl.pallas_call(
        _scale_body,
        grid=(_GRID,),
        in_specs=[pl.BlockSpec((_ROW_BLK, HID), lambda i: (i, 0)),
                  pl.BlockSpec((NC, _ROW_BLK, DEG_W), lambda i: (0, i, 0))],
        out_specs=pl.BlockSpec((_ROW_BLK, GATHER_W), lambda i: (i, 0)),
        out_shape=jax.ShapeDtypeStruct((N_PAD, GATHER_W), jnp.float32),
    )(h, degp)


def _sc_scatter(src3, dst3, hp_pad, zeros_init):
    chunks = src3.shape[1]

    NBUF = 2

    @functools.partial(
        pl.kernel,
        out_type=jax.ShapeDtypeStruct((NC, N_PAD, GATHER_W), jnp.float32),
        mesh=_sc_mesh,
        scratch_types=[
            pltpu.VMEM((chunks, K), jnp.int32),
            pltpu.VMEM((chunks, K), jnp.int32),
        ] + [pltpu.VMEM((K, GATHER_W), jnp.float32) for _ in range(NBUF)]
          + [pltpu.VMEM_SHARED((N_PAD, GATHER_W), jnp.float32)]
          + [pltpu.SemaphoreType.DMA for _ in range(NBUF)],
    )
    def k(src_hbm, dst_hbm, hp_hbm, zero_hbm, out_hbm,
          src_v, dst_v, *rest):
        rows = rest[:NBUF]
        acc_sh = rest[NBUF]
        sems = rest[NBUF + 1:]
        cid = lax.axis_index("c")
        sid = lax.axis_index("s")
        tid = cid * NS + sid
        pltpu.sync_copy(src_hbm.at[tid], src_v)
        pltpu.sync_copy(dst_hbm.at[tid], dst_v)
        pltpu.sync_copy(zero_hbm, acc_sh.at[pl.ds(sid * ROWS_PER_TILE,
                                                  ROWS_PER_TILE)])
        plsc.subcore_barrier()

        for p in range(NBUF - 1):
            pltpu.async_copy(hp_hbm.at[src_v.at[p]], rows[p], sems[p])

        @pl.loop(0, chunks, step=NBUF)
        def _(j):
            for b in range(NBUF):
                jb = j + b
                nxt = (b + NBUF - 1) % NBUF

                @pl.when(jb + NBUF - 1 < chunks)
                def _():
                    pltpu.async_copy(hp_hbm.at[src_v.at[jb + NBUF - 1]],
                                     rows[nxt], sems[nxt])

                pltpu.make_async_copy(hp_hbm.at[src_v.at[jb]],
                                      rows[b], sems[b]).wait()
                pltpu.sync_copy(rows[b], acc_sh.at[dst_v.at[jb]], add=True)

        plsc.subcore_barrier()
        pltpu.sync_copy(
            acc_sh.at[pl.ds(sid * ROWS_PER_TILE, ROWS_PER_TILE)],
            out_hbm.at[cid, pl.ds(sid * ROWS_PER_TILE, ROWS_PER_TILE)],
        )

    return k(src3, dst3, hp_pad, zeros_init)


def _final_body(h_ref, deg_ref, acc_ref, bg_ref,
                wm_ref, bm_ref, wl_ref, bl_ref, mu_ref, lv_ref, sacc):
    i = pl.program_id(0)
    dis = lax.rsqrt(deg_ref[0, :, 0:1] + deg_ref[1, :, 0:1] + 1.0)
    pre = (dis * (acc_ref[0, :, :HID] + acc_ref[1, :, :HID])
           + dis * dis * h_ref[...] + bg_ref[...])
    psum = jnp.sum(jnp.maximum(pre, 0.0), axis=0, keepdims=True)

    @pl.when(i == 0)
    def _():
        sacc[...] = psum

    @pl.when(i > 0)
    def _():
        sacc[...] += psum

    @pl.when(i == _GRID - 1)
    def _():
        pooled = sacc[...] * (1.0 / N)
        mu_ref[...] = jnp.dot(pooled, wm_ref[...],
                              preferred_element_type=jnp.float32) + bm_ref[...]
        lv_ref[...] = jnp.dot(pooled, wl_ref[...],
                              preferred_element_type=jnp.float32) + bl_ref[...]


def _tc_final(h, degp, accp, bg, wm, bm, wl, bl):
    return pl.pallas_call(
        _final_body,
        grid=(_GRID,),
        in_specs=[pl.BlockSpec((_ROW_BLK, HID), lambda i: (i, 0)),
                  pl.BlockSpec((NC, _ROW_BLK, DEG_W), lambda i: (0, i, 0)),
                  pl.BlockSpec((NC, _ROW_BLK, GATHER_W), lambda i: (0, i, 0)),
                  pl.BlockSpec((1, HID), lambda i: (0, 0)),
                  pl.BlockSpec((HID, LATENT), lambda i: (0, 0)),
                  pl.BlockSpec((1, LATENT), lambda i: (0, 0)),
                  pl.BlockSpec((HID, LATENT), lambda i: (0, 0)),
                  pl.BlockSpec((1, LATENT), lambda i: (0, 0))],
        out_specs=[pl.BlockSpec((1, LATENT), lambda i: (0, 0)),
                   pl.BlockSpec((1, LATENT), lambda i: (0, 0))],
        out_shape=[jax.ShapeDtypeStruct((1, LATENT), jnp.float32),
                   jax.ShapeDtypeStruct((1, LATENT), jnp.float32)],
        scratch_shapes=[pltpu.VMEM((1, HID), jnp.float32)],
    )(h, degp, accp, bg, wm, bm, wl, bl)


def kernel(x, edge_index, W_gcn, b_gcn, W_mu, b_mu, W_lv, b_lv):
    e = edge_index.shape[1]
    e1 = ((e + NW - 1) // NW) * NW
    pw = e1 // NW
    chunks = (pw + K - 1) // K
    cols_pad = chunks * K - pw

    tail1 = jnp.full((e1 - e,), DUMMY, jnp.int32)
    src2 = jnp.concatenate([edge_index[0], tail1]).reshape(NW, pw)
    dst2 = jnp.concatenate([edge_index[1], tail1]).reshape(NW, pw)
    dummies = (DUMMY + (jnp.arange(NW * cols_pad, dtype=jnp.int32)
                        % (N_PAD - N))).reshape(NW, cols_pad)
    src3 = jnp.concatenate([src2, dummies], axis=1).reshape(NW, chunks, K)
    dst3 = jnp.concatenate([dst2, dummies], axis=1).reshape(NW, chunks, K)

    ones_rows = jnp.ones((K, DEG_W), jnp.float32)
    zerosd = jnp.zeros((ROWS_PER_TILE, DEG_W), jnp.float32)
    zerosw = jnp.zeros((ROWS_PER_TILE, GATHER_W), jnp.float32)

    h = _tc_matmul(x, W_gcn)
    degp = _sc_degree(dst3, ones_rows, zerosd)
    hp_pad = _tc_scale(h, degp)
    accp = _sc_scatter(src3, dst3, hp_pad, zerosw)
    mu2, lv2 = _tc_final(h, degp, accp,
                         b_gcn.reshape(1, HID), W_mu, b_mu.reshape(1, LATENT),
                         W_lv, b_lv.reshape(1, LATENT))
    return mu2.reshape(LATENT), lv2.reshape(LATENT)
